# Initial kernel scaffold; baseline (speedup 1.0000x reference)
import jax
import jax.numpy as jnp
from jax.experimental import pallas as pl


def kernel(node_embeddings, W1, a_src1, a_dst1, b1, W2, a_src2, a_dst2, b2):
    raise NotImplementedError("write your pallas kernel here")


# speedup vs baseline: 83.7170x; 83.7170x over previous
import functools

import jax
import jax.numpy as jnp
from jax.experimental import pallas as pl


def _gat_block_kernel(xc_ref, xt_ref, xb_ref, w_ref, b_ref, o_ref, *, R, S, H, C):
    Din = xc_ref.shape[3]
    HC = H * C
    n0 = 2 * S

    x_ext = jnp.concatenate(
        [
            xt_ref[0].reshape(2 * S, Din),
            xc_ref[0].reshape(R * S, Din),
            xb_ref[0].reshape(2 * S, Din),
        ],
        axis=0,
    )

    y = jax.lax.dot_general(
        x_ext, w_ref[...],
        (((1,), (0,)), ((), ())),
        preferred_element_type=jnp.float32,
    )

    def csl(o):
        return y[n0 + o : n0 + o + R * S, HC : HC + H]

    asrc_self = csl(0)
    adst = y[n0 : n0 + R * S, HC + H : HC + 2 * H]

    n = jax.lax.broadcasted_iota(jnp.int32, (R * S, 1), 0)
    ti = pl.program_id(1) * R + n // S
    tj = n % S

    def lrelu(v):
        return jnp.where(v >= 0, v, 0.2 * v)

    NEG = jnp.float32(-1e30)
    v1 = (ti >= 1) & (tj >= 1) & (ti <= S - 2) & (tj <= S - 2) & (tj > ti)
    v2 = (ti >= 1) & (tj >= 1) & (tj > ti)
    v3 = (ti >= 1) & (tj >= ti + 2)
    v4 = (ti >= 1) & (ti <= S - 2) & (tj >= ti + 2)

    OFF1, OFF2, OFF3, OFF4 = S + 1, -(S + 1), -1, S
    e_self = lrelu(asrc_self + adst)
    e1 = jnp.where(v1, lrelu(csl(OFF1) + adst), NEG)
    e2 = jnp.where(v2, lrelu(csl(OFF2) + adst), NEG)
    e3 = jnp.where(v3, lrelu(csl(OFF3) + adst), NEG)
    e4 = jnp.where(v4, lrelu(csl(OFF4) + adst), NEG)

    m = jnp.maximum(e_self, jnp.maximum(jnp.maximum(e1, e2), jnp.maximum(e3, e4)))
    w_self = jnp.exp(e_self - m)
    w1 = jnp.exp(e1 - m)
    w2 = jnp.exp(e2 - m)
    w3 = jnp.exp(e3 - m)
    w4 = jnp.exp(e4 - m)
    scale = 1.0 / (H * (w_self + w1 + w2 + w3 + w4))
    w_self = w_self * scale
    w1 = w1 * scale
    w2 = w2 * scale
    w3 = w3 * scale
    w4 = w4 * scale

    acc = jnp.zeros((R * S, C), jnp.float32)
    for hh in range(H):
        cs = slice(hh * C, (hh + 1) * C)

        def hsl(o):
            return y[n0 + o : n0 + o + R * S, cs]

        acc += hsl(0) * w_self[:, hh : hh + 1]
        acc += hsl(OFF1) * w1[:, hh : hh + 1]
        acc += hsl(OFF2) * w2[:, hh : hh + 1]
        acc += hsl(OFF3) * w3[:, hh : hh + 1]
        acc += hsl(OFF4) * w4[:, hh : hh + 1]

    o_ref[0] = (acc + b_ref[...]).reshape(R, S, C)


def _gat_layer(x, w_aug, b, H, C, R):
    B, S, _, Din = x.shape
    G = S // R
    waug_cols = w_aug.shape[1]
    kern = functools.partial(_gat_block_kernel, R=R, S=S, H=H, C=C)
    return pl.pallas_call(
        kern,
        grid=(B, G),
        in_specs=[
            pl.BlockSpec((1, R, S, Din), lambda b_, g: (b_, g, 0, 0)),
            pl.BlockSpec(
                (1, 2, S, Din),
                lambda b_, g: (b_, jnp.maximum(g * (R // 2) - 1, 0), 0, 0),
            ),
            pl.BlockSpec(
                (1, 2, S, Din),
                lambda b_, g: (b_, jnp.minimum((g + 1) * (R // 2), S // 2 - 1), 0, 0),
            ),
            pl.BlockSpec((Din, waug_cols), lambda b_, g: (0, 0)),
            pl.BlockSpec((1, C), lambda b_, g: (0, 0)),
        ],
        out_specs=pl.BlockSpec((1, R, S, C), lambda b_, g: (b_, g, 0, 0)),
        out_shape=jax.ShapeDtypeStruct((B, S, S, C), jnp.float32),
    )(x, x, x, w_aug, b.reshape(1, C))


def _augment_weights(W, a_src, a_dst):
    Din = W.shape[0]
    H, C = a_src.shape
    Wr = W.reshape(Din, H, C)
    ws = jnp.einsum("dhc,hc->dh", Wr, a_src)
    wd = jnp.einsum("dhc,hc->dh", Wr, a_dst)
    pad = jnp.zeros((Din, 128 - 2 * H), W.dtype)
    return jnp.concatenate([W, ws, wd, pad], axis=1)


def kernel(node_embeddings, W1, a_src1, a_dst1, b1, W2, a_src2, a_dst2, b2):
    H, C = a_src1.shape
    R = 16
    w1 = _augment_weights(W1, a_src1, a_dst1)
    w2 = _augment_weights(W2, a_src2, a_dst2)
    out1 = _gat_layer(node_embeddings, w1, b1, H, C, R)
    out2 = _gat_layer(out1, w2, b2, H, C, R)
    return out2


# speedup vs baseline: 130.0523x; 1.5535x over previous
import functools

import jax
import jax.numpy as jnp
from jax.experimental import pallas as pl


def _gat_block_kernel(xc_ref, xt_ref, xb_ref, w_ref, b_ref, o_ref, *, R, S, H, C):
    Din = xc_ref.shape[3]
    HC = H * C
    RS = R * S
    n0 = 2 * S

    x_ext = jnp.concatenate(
        [
            xt_ref[0].reshape(2 * S, Din),
            xc_ref[0].reshape(RS, Din),
            xb_ref[0].reshape(2 * S, Din),
        ],
        axis=0,
    )

    y = jax.lax.dot_general(
        x_ext, w_ref[...],
        (((1,), (0,)), ((), ())),
        preferred_element_type=jnp.float32,
    )

    al_t = jnp.swapaxes(y[:, HC : HC + 2 * H], 0, 1)
    adst = al_t[H : 2 * H, n0 : n0 + RS]

    def asl(o):
        return al_t[0:H, n0 + o : n0 + o + RS]

    n = jax.lax.broadcasted_iota(jnp.int32, (1, RS), 1)
    ti = pl.program_id(1) * R + n // S
    tj = n % S

    def lrelu(v):
        return jnp.where(v >= 0, v, 0.2 * v)

    NEG = jnp.float32(-1e30)
    v1 = (ti >= 1) & (tj >= 1) & (ti <= S - 2) & (tj <= S - 2) & (tj > ti)
    v2 = (ti >= 1) & (tj >= 1) & (tj > ti)
    v3 = (ti >= 1) & (tj >= ti + 2)
    v4 = (ti >= 1) & (ti <= S - 2) & (tj >= ti + 2)

    OFF1, OFF2, OFF3, OFF4 = S + 1, -(S + 1), -1, S
    e_self = lrelu(asl(0) + adst)
    e1 = jnp.where(v1, lrelu(asl(OFF1) + adst), NEG)
    e2 = jnp.where(v2, lrelu(asl(OFF2) + adst), NEG)
    e3 = jnp.where(v3, lrelu(asl(OFF3) + adst), NEG)
    e4 = jnp.where(v4, lrelu(asl(OFF4) + adst), NEG)

    m = jnp.maximum(e_self, jnp.maximum(jnp.maximum(e1, e2), jnp.maximum(e3, e4)))
    w_self = jnp.exp(e_self - m)
    w1 = jnp.exp(e1 - m)
    w2 = jnp.exp(e2 - m)
    w3 = jnp.exp(e3 - m)
    w4 = jnp.exp(e4 - m)
    scale = 1.0 / (H * (w_self + w1 + w2 + w3 + w4))

    w5 = jnp.swapaxes(
        jnp.concatenate(
            [w_self * scale, w1 * scale, w2 * scale, w3 * scale, w4 * scale], axis=0
        ),
        0,
        1,
    )

    pg = jax.lax.broadcasted_iota(jnp.int32, (5 * H, 5 * HC), 0)
    pj = jax.lax.broadcasted_iota(jnp.int32, (5 * H, 5 * HC), 1)
    P5 = ((pj // HC == pg // H) & ((pj % HC) // C == pg % H)).astype(jnp.float32)

    offs = (0, OFF1, OFF2, OFF3, OFF4)
    CH = 128
    for r in range(0, RS, CH):
        w_exp = jax.lax.dot_general(
            w5[r : r + CH, :], P5,
            (((1,), (0,)), ((), ())),
            preferred_element_type=jnp.float32,
        )
        acc = None
        for k, o in enumerate(offs):
            term = y[n0 + o + r : n0 + o + r + CH, 0:HC] * w_exp[:, k * HC : (k + 1) * HC]
            acc = term if acc is None else acc + term
        out = b_ref[...]
        for hh in range(H):
            out = out + acc[:, hh * C : (hh + 1) * C]
        o_ref[0, r // S : r // S + CH // S] = out.reshape(CH // S, S, C)


def _gat_layer(x, w_aug, b, H, C, R):
    B, S, _, Din = x.shape
    G = S // R
    waug_cols = w_aug.shape[1]
    kern = functools.partial(_gat_block_kernel, R=R, S=S, H=H, C=C)
    return pl.pallas_call(
        kern,
        grid=(B, G),
        in_specs=[
            pl.BlockSpec((1, R, S, Din), lambda b_, g: (b_, g, 0, 0)),
            pl.BlockSpec(
                (1, 2, S, Din),
                lambda b_, g: (b_, jnp.maximum(g * (R // 2) - 1, 0), 0, 0),
            ),
            pl.BlockSpec(
                (1, 2, S, Din),
                lambda b_, g: (b_, jnp.minimum((g + 1) * (R // 2), S // 2 - 1), 0, 0),
            ),
            pl.BlockSpec((Din, waug_cols), lambda b_, g: (0, 0)),
            pl.BlockSpec((1, C), lambda b_, g: (0, 0)),
        ],
        out_specs=pl.BlockSpec((1, R, S, C), lambda b_, g: (b_, g, 0, 0)),
        out_shape=jax.ShapeDtypeStruct((B, S, S, C), jnp.float32),
    )(x, x, x, w_aug, b.reshape(1, C))


def _augment_weights(W, a_src, a_dst):
    Din = W.shape[0]
    H, C = a_src.shape
    Wr = W.reshape(Din, H, C)
    ws = jnp.einsum("dhc,hc->dh", Wr, a_src)
    wd = jnp.einsum("dhc,hc->dh", Wr, a_dst)
    pad = jnp.zeros((Din, 128 - 2 * H), W.dtype)
    return jnp.concatenate([W, ws, wd, pad], axis=1)


def kernel(node_embeddings, W1, a_src1, a_dst1, b1, W2, a_src2, a_dst2, b2):
    H, C = a_src1.shape
    R = 16
    w1 = _augment_weights(W1, a_src1, a_dst1)
    w2 = _augment_weights(W2, a_src2, a_dst2)
    out1 = _gat_layer(node_embeddings, w1, b1, H, C, R)
    out2 = _gat_layer(out1, w2, b2, H, C, R)
    return out2


# speedup vs baseline: 137.2008x; 1.0550x over previous
import functools

import jax
import jax.numpy as jnp
from jax.experimental import pallas as pl


def _gat_block_kernel(xc_ref, xt_ref, xb_ref, w_ref, b_ref, o_ref, *, R, S, H, C):
    Din = xc_ref.shape[3]
    HC = H * C
    RS = R * S
    n0 = 2 * S

    x_ext = jnp.concatenate(
        [
            xt_ref[0].reshape(2 * S, Din),
            xc_ref[0].reshape(RS, Din),
            xb_ref[0].reshape(2 * S, Din),
        ],
        axis=0,
    )

    y = jax.lax.dot_general(
        x_ext, w_ref[...],
        (((1,), (0,)), ((), ())),
        preferred_element_type=jnp.float32,
    )

    al_t = jnp.swapaxes(y[:, HC : HC + 2 * H], 0, 1)
    adst = al_t[H : 2 * H, n0 : n0 + RS]

    def asl(o):
        return al_t[0:H, n0 + o : n0 + o + RS]

    n = jax.lax.broadcasted_iota(jnp.int32, (1, RS), 1)
    ti = pl.program_id(1) * R + n // S
    tj = n % S

    def lrelu(v):
        return jnp.where(v >= 0, v, 0.2 * v)

    NEG = jnp.float32(-1e30)
    v1 = (ti >= 1) & (tj >= 1) & (ti <= S - 2) & (tj <= S - 2) & (tj > ti)
    v2 = (ti >= 1) & (tj >= 1) & (tj > ti)
    v3 = (ti >= 1) & (tj >= ti + 2)
    v4 = (ti >= 1) & (ti <= S - 2) & (tj >= ti + 2)

    OFF1, OFF2, OFF3, OFF4 = S + 1, -(S + 1), -1, S
    e_self = lrelu(asl(0) + adst)
    e1 = jnp.where(v1, lrelu(asl(OFF1) + adst), NEG)
    e2 = jnp.where(v2, lrelu(asl(OFF2) + adst), NEG)
    e3 = jnp.where(v3, lrelu(asl(OFF3) + adst), NEG)
    e4 = jnp.where(v4, lrelu(asl(OFF4) + adst), NEG)

    m = jnp.maximum(e_self, jnp.maximum(jnp.maximum(e1, e2), jnp.maximum(e3, e4)))
    w_self = jnp.exp(e_self - m)
    w1 = jnp.exp(e1 - m)
    w2 = jnp.exp(e2 - m)
    w3 = jnp.exp(e3 - m)
    w4 = jnp.exp(e4 - m)
    scale = 1.0 / (H * (w_self + w1 + w2 + w3 + w4))

    w5 = jnp.swapaxes(
        jnp.concatenate(
            [w_self * scale, w1 * scale, w2 * scale, w3 * scale, w4 * scale], axis=0
        ),
        0,
        1,
    )

    pg = jax.lax.broadcasted_iota(jnp.int32, (5 * H, 5 * HC), 0)
    pj = jax.lax.broadcasted_iota(jnp.int32, (5 * H, 5 * HC), 1)
    P5 = ((pj // HC == pg // H) & ((pj % HC) // C == pg % H)).astype(jnp.float32)

    offs = (0, OFF1, OFF2, OFF3, OFF4)
    CH = 128
    for r in range(0, RS, CH):
        w_exp = jax.lax.dot_general(
            w5[r : r + CH, :], P5,
            (((1,), (0,)), ((), ())),
            preferred_element_type=jnp.float32,
        )
        acc = None
        for k, o in enumerate(offs):
            term = y[n0 + o + r : n0 + o + r + CH, 0:HC] * w_exp[:, k * HC : (k + 1) * HC]
            acc = term if acc is None else acc + term
        out = b_ref[...]
        for hh in range(H):
            out = out + acc[:, hh * C : (hh + 1) * C]
        o_ref[0, r // S : r // S + CH // S] = out.reshape(CH // S, S, C)


def _gat_layer(x, w_aug, b, H, C, R):
    B, S, _, Din = x.shape
    G = S // R
    waug_cols = w_aug.shape[1]
    kern = functools.partial(_gat_block_kernel, R=R, S=S, H=H, C=C)
    return pl.pallas_call(
        kern,
        grid=(B, G),
        in_specs=[
            pl.BlockSpec((1, R, S, Din), lambda b_, g: (b_, g, 0, 0)),
            pl.BlockSpec(
                (1, 2, S, Din),
                lambda b_, g: (b_, jnp.maximum(g * (R // 2) - 1, 0), 0, 0),
            ),
            pl.BlockSpec(
                (1, 2, S, Din),
                lambda b_, g: (b_, jnp.minimum((g + 1) * (R // 2), S // 2 - 1), 0, 0),
            ),
            pl.BlockSpec((Din, waug_cols), lambda b_, g: (0, 0)),
            pl.BlockSpec((1, C), lambda b_, g: (0, 0)),
        ],
        out_specs=pl.BlockSpec((1, R, S, C), lambda b_, g: (b_, g, 0, 0)),
        out_shape=jax.ShapeDtypeStruct((B, S, S, C), jnp.float32),
    )(x, x, x, w_aug, b.reshape(1, C))


def _augment_weights(W, a_src, a_dst):
    Din = W.shape[0]
    H, C = a_src.shape
    Wr = W.reshape(Din, H, C)
    ws = jnp.einsum("dhc,hc->dh", Wr, a_src)
    wd = jnp.einsum("dhc,hc->dh", Wr, a_dst)
    pad = jnp.zeros((Din, 128 - 2 * H), W.dtype)
    return jnp.concatenate([W, ws, wd, pad], axis=1)


def kernel(node_embeddings, W1, a_src1, a_dst1, b1, W2, a_src2, a_dst2, b2):
    H, C = a_src1.shape
    R = 32
    w1 = _augment_weights(W1, a_src1, a_dst1)
    w2 = _augment_weights(W2, a_src2, a_dst2)
    out1 = _gat_layer(node_embeddings, w1, b1, H, C, R)
    out2 = _gat_layer(out1, w2, b2, H, C, R)
    return out2


# speedup vs baseline: 195.0587x; 1.4217x over previous
import functools

import jax
import jax.numpy as jnp
from jax.experimental import pallas as pl
from jax.experimental.pallas import tpu as pltpu


def _gat_block_kernel(xc_ref, xt_ref, xb_ref, w_ref, b_ref, o_ref, ym_ref, yp_ref, *, R, S, H, C):
    Din = xc_ref.shape[3]
    HC = H * C
    RS = R * S
    n0 = 2 * S

    x_ext = jnp.concatenate(
        [
            xt_ref[0].reshape(2 * S, Din),
            xc_ref[0].reshape(RS, Din),
            xb_ref[0].reshape(2 * S, Din),
        ],
        axis=0,
    )

    y = jax.lax.dot_general(
        x_ext, w_ref[...],
        (((1,), (0,)), ((), ())),
        preferred_element_type=jnp.float32,
    )

    al_t = jnp.swapaxes(y[:, HC : HC + 2 * H], 0, 1)
    adst = al_t[H : 2 * H, n0 : n0 + RS]

    def asl(o):
        return al_t[0:H, n0 + o : n0 + o + RS]

    n = jax.lax.broadcasted_iota(jnp.int32, (1, RS), 1)
    ti = pl.program_id(1) * R + n // S
    tj = n % S

    def lrelu(v):
        return jnp.where(v >= 0, v, 0.2 * v)

    NEG = jnp.float32(-1e30)
    v1 = (ti >= 1) & (tj >= 1) & (ti <= S - 2) & (tj <= S - 2) & (tj > ti)
    v2 = (ti >= 1) & (tj >= 1) & (tj > ti)
    v3 = (ti >= 1) & (tj >= ti + 2)
    v4 = (ti >= 1) & (ti <= S - 2) & (tj >= ti + 2)

    OFF1, OFF2, OFF3, OFF4 = S + 1, -(S + 1), -1, S
    e_self = lrelu(asl(0) + adst)
    e1 = jnp.where(v1, lrelu(asl(OFF1) + adst), NEG)
    e2 = jnp.where(v2, lrelu(asl(OFF2) + adst), NEG)
    e3 = jnp.where(v3, lrelu(asl(OFF3) + adst), NEG)
    e4 = jnp.where(v4, lrelu(asl(OFF4) + adst), NEG)

    m = jnp.maximum(e_self, jnp.maximum(jnp.maximum(e1, e2), jnp.maximum(e3, e4)))
    w_self = jnp.exp(e_self - m)
    w1 = jnp.exp(e1 - m)
    w2 = jnp.exp(e2 - m)
    w3 = jnp.exp(e3 - m)
    w4 = jnp.exp(e4 - m)
    scale = 1.0 / (H * (w_self + w1 + w2 + w3 + w4))

    w5 = jnp.swapaxes(
        jnp.concatenate(
            [w_self * scale, w1 * scale, w2 * scale, w3 * scale, w4 * scale], axis=0
        ),
        0,
        1,
    )

    pg = jax.lax.broadcasted_iota(jnp.int32, (5 * H, 5 * HC), 0)
    pj = jax.lax.broadcasted_iota(jnp.int32, (5 * H, 5 * HC), 1)
    P5 = ((pj // HC == pg // H) & ((pj % HC) // C == pg % H)).astype(jnp.bfloat16)
    w5b = w5.astype(jnp.bfloat16)

    EXT = (R + 4) * S
    ym_ref[...] = y[7 : EXT - 8 + 7, 0:HC]
    yp_ref[...] = y[1 : EXT - 8 + 1, 0:HC]

    CH = 128
    for r in range(0, RS, CH):
        w_exp = jax.lax.dot_general(
            w5b[r : r + CH, :], P5,
            (((1,), (0,)), ((), ())),
            preferred_element_type=jnp.float32,
        )
        slices = (
            y[n0 + r : n0 + r + CH, 0:HC],
            yp_ref[n0 + S + r : n0 + S + r + CH, :],
            ym_ref[n0 - S - 8 + r : n0 - S - 8 + r + CH, :],
            ym_ref[n0 - 8 + r : n0 - 8 + r + CH, :],
            y[n0 + S + r : n0 + S + r + CH, 0:HC],
        )
        acc = None
        for k, sl in enumerate(slices):
            term = sl * w_exp[:, k * HC : (k + 1) * HC]
            acc = term if acc is None else acc + term
        out = b_ref[...]
        for hh in range(H):
            out = out + acc[:, hh * C : (hh + 1) * C]
        o_ref[0, r // S : r // S + CH // S] = out.reshape(CH // S, S, C)


def _gat_layer(x, w_aug, b, H, C, R):
    B, S, _, Din = x.shape
    G = S // R
    waug_cols = w_aug.shape[1]
    kern = functools.partial(_gat_block_kernel, R=R, S=S, H=H, C=C)
    return pl.pallas_call(
        kern,
        grid=(B, G),
        in_specs=[
            pl.BlockSpec((1, R, S, Din), lambda b_, g: (b_, g, 0, 0)),
            pl.BlockSpec(
                (1, 2, S, Din),
                lambda b_, g: (b_, jnp.maximum(g * (R // 2) - 1, 0), 0, 0),
            ),
            pl.BlockSpec(
                (1, 2, S, Din),
                lambda b_, g: (b_, jnp.minimum((g + 1) * (R // 2), S // 2 - 1), 0, 0),
            ),
            pl.BlockSpec((Din, waug_cols), lambda b_, g: (0, 0)),
            pl.BlockSpec((1, C), lambda b_, g: (0, 0)),
        ],
        out_specs=pl.BlockSpec((1, R, S, C), lambda b_, g: (b_, g, 0, 0)),
        out_shape=jax.ShapeDtypeStruct((B, S, S, C), jnp.float32),
        scratch_shapes=[
            pltpu.VMEM(((R + 4) * S - 8, H * C), jnp.float32),
            pltpu.VMEM(((R + 4) * S - 8, H * C), jnp.float32),
        ],
    )(x, x, x, w_aug, b.reshape(1, C))


def _augment_weights(W, a_src, a_dst):
    Din = W.shape[0]
    H, C = a_src.shape
    Wr = W.reshape(Din, H, C)
    ws = jnp.einsum("dhc,hc->dh", Wr, a_src)
    wd = jnp.einsum("dhc,hc->dh", Wr, a_dst)
    pad = jnp.zeros((Din, 128 - 2 * H), W.dtype)
    return jnp.concatenate([W, ws, wd, pad], axis=1)


def kernel(node_embeddings, W1, a_src1, a_dst1, b1, W2, a_src2, a_dst2, b2):
    H, C = a_src1.shape
    R = 32
    w1 = _augment_weights(W1, a_src1, a_dst1)
    w2 = _augment_weights(W2, a_src2, a_dst2)
    out1 = _gat_layer(node_embeddings, w1, b1, H, C, R)
    out2 = _gat_layer(out1, w2, b2, H, C, R)
    return out2
